# baseline (speedup 1.0000x reference)
import jax
import jax.numpy as jnp
from jax import lax
from jax.experimental import pallas as pl

_OUT = 128
_MAX = 47
_SUM = 47
_RND = _OUT - 2 * _MAX
_S = 196
_POOL2 = _S - _MAX
_POOL3 = _POOL2 - _SUM


def _build_q():
    idx_lucky = jax.random.permutation(jax.random.key(42), _POOL3)[:_RND]
    idx43 = jax.random.permutation(jax.random.key(43), _OUT)
    inv43 = jnp.zeros(_OUT, jnp.int32).at[idx43].set(jnp.arange(_OUT, dtype=jnp.int32))
    invlucky = jnp.full(_POOL3, _OUT, jnp.int32).at[idx_lucky].set(
        jnp.arange(_RND, dtype=jnp.int32))
    kept = invlucky < _RND
    t3 = jnp.where(kept, inv43[jnp.clip(2 * _MAX + invlucky, 0, _OUT - 1)], 999)
    t = jnp.concatenate([inv43[: 2 * _MAX], t3])
    q = (t[:, None] == jnp.arange(_OUT, dtype=jnp.int32)[None, :]).astype(jnp.float32)
    return q


def _body(route_ref, votes_ref, q_ref, out_ref):
    r = route_ref[0, 0]
    rmax = jnp.max(r, axis=0)
    rsum = jnp.sum(r, axis=0)

    it = lax.broadcasted_iota(jnp.int32, (_S, _S), 1)
    is_ = lax.broadcasted_iota(jnp.int32, (_S, _S), 0)

    xt = rmax[None, :]
    xs = rmax[:, None]
    m1mat = (xt > xs) | ((xt == xs) & (it < is_))
    r1 = jnp.sum(m1mat.astype(jnp.float32), axis=1)

    pool_t = (r1 >= _MAX)[None, :]
    yt = rsum[None, :]
    ys = rsum[:, None]
    r1t = r1[None, :]
    r1s = r1[:, None]
    m2mat = pool_t & ((yt > ys) | ((yt == ys) & (r1t < r1s)))
    r2 = jnp.sum(m2mat.astype(jnp.float32), axis=1)

    c = jnp.where(r1 < _MAX, r1, _MAX + r2)
    cmat = (c[:, None] == lax.broadcasted_iota(jnp.int32, (_S, _S), 1).astype(
        jnp.float32)).astype(jnp.float32)
    p = jnp.dot(cmat, q_ref[...], preferred_element_type=jnp.float32)

    v = votes_ref[0, 0]
    out = lax.dot_general(v, p, (((2,), (0,)), ((), ())),
                          preferred_element_type=jnp.float32)
    out_ref[0, 0] = out


def kernel(votes, route):
    b, input_dim, output_dim, h = votes.shape[:4]
    votes = votes.reshape(b, input_dim, output_dim, h, -1)
    route = route.reshape(b, input_dim, output_dim, -1)
    q = _build_q()

    out = pl.pallas_call(
        _body,
        grid=(b, input_dim),
        in_specs=[
            pl.BlockSpec((1, 1, output_dim, _S), lambda bi, ii: (bi, ii, 0, 0)),
            pl.BlockSpec((1, 1, output_dim, h, _S), lambda bi, ii: (bi, ii, 0, 0, 0)),
            pl.BlockSpec((_S, _OUT), lambda bi, ii: (0, 0)),
        ],
        out_specs=pl.BlockSpec((1, 1, output_dim, h, _OUT),
                               lambda bi, ii: (bi, ii, 0, 0, 0)),
        out_shape=jax.ShapeDtypeStruct((b, input_dim, output_dim, h, _OUT),
                                       jnp.float32),
    )(route, votes, q)
    return out[..., None]


# speedup vs baseline: 5.5785x; 5.5785x over previous
import jax
import jax.numpy as jnp
from jax import lax
from jax.experimental import pallas as pl

_OUT = 128
_MAX = 47
_SUM = 47
_RND = _OUT - 2 * _MAX
_S = 196
_POOL2 = _S - _MAX
_POOL3 = _POOL2 - _SUM


def _build_q():
    idx_lucky = jax.random.permutation(jax.random.key(42), _POOL3)[:_RND]
    idx43 = jax.random.permutation(jax.random.key(43), _OUT)
    inv43 = jnp.zeros(_OUT, jnp.int32).at[idx43].set(jnp.arange(_OUT, dtype=jnp.int32))
    invlucky = jnp.full(_POOL3, _OUT, jnp.int32).at[idx_lucky].set(
        jnp.arange(_RND, dtype=jnp.int32))
    kept = invlucky < _RND
    t3 = jnp.where(kept, inv43[jnp.clip(2 * _MAX + invlucky, 0, _OUT - 1)], 999)
    t = jnp.concatenate([inv43[: 2 * _MAX], t3])
    q = (t[:, None] == jnp.arange(_OUT, dtype=jnp.int32)[None, :]).astype(jnp.float32)
    return q


def _mm(a, b):
    return jnp.dot(a, b, preferred_element_type=jnp.float32)


def _body(route_ref, votes_ref, q_ref, out_ref):
    r = route_ref[0, 0]
    ones_col = jnp.ones((_S, 1), jnp.float32)
    ones_row = jnp.ones((1, _S), jnp.float32)
    i0 = lax.broadcasted_iota(jnp.int32, (_S, _S), 0)
    i1 = lax.broadcasted_iota(jnp.int32, (_S, _S), 1)
    eye = (i0 == i1).astype(jnp.float32)

    def tocol(v_row):
        return _mm(v_row * eye, ones_col)

    x_row = jnp.max(r, axis=0, keepdims=True)
    y_row = jnp.sum(r, axis=0, keepdims=True)
    x_cb = _mm(tocol(x_row), ones_row)
    y_cb = _mm(tocol(y_row), ones_row)

    m1 = jnp.where((x_cb > x_row) | ((x_cb == x_row) & (i0 < i1)), 1.0, 0.0)
    r1_row = _mm(ones_row, m1)
    r1_cb = _mm(tocol(r1_row), ones_row)

    pool_cb = r1_cb >= _MAX
    m2 = jnp.where(
        pool_cb & ((y_cb > y_row) | ((y_cb == y_row) & (r1_cb < r1_row))),
        1.0, 0.0)
    r2_row = _mm(ones_row, m2)

    c_row = jnp.where(r1_row < _MAX, r1_row, _MAX + r2_row)
    c_cb = _mm(tocol(c_row), ones_row)
    cmat = (c_cb == i1.astype(jnp.float32)).astype(jnp.float32)
    p = _mm(cmat, q_ref[...])

    v = votes_ref[0, 0]
    out = lax.dot_general(v, p, (((2,), (0,)), ((), ())),
                          preferred_element_type=jnp.float32)
    out_ref[0, 0] = out


def kernel(votes, route):
    b, input_dim, output_dim, h = votes.shape[:4]
    votes = votes.reshape(b, input_dim, output_dim, h, -1)
    route = route.reshape(b, input_dim, output_dim, -1)
    q = _build_q()

    out = pl.pallas_call(
        _body,
        grid=(b, input_dim),
        in_specs=[
            pl.BlockSpec((1, 1, output_dim, _S), lambda bi, ii: (bi, ii, 0, 0)),
            pl.BlockSpec((1, 1, output_dim, h, _S), lambda bi, ii: (bi, ii, 0, 0, 0)),
            pl.BlockSpec((_S, _OUT), lambda bi, ii: (0, 0)),
        ],
        out_specs=pl.BlockSpec((1, 1, output_dim, h, _OUT),
                               lambda bi, ii: (bi, ii, 0, 0, 0)),
        out_shape=jax.ShapeDtypeStruct((b, input_dim, output_dim, h, _OUT),
                                       jnp.float32),
    )(route, votes, q)
    return out[..., None]
